# baseline (speedup 1.0000x reference)
import functools

import jax
import jax.numpy as jnp
from jax import lax
from jax.experimental import pallas as pl
from jax.experimental.pallas import tpu as pltpu
from jax.experimental.pallas import tpu_sc as plsc

EMB = 64
NC = 2
NS = 16
NW = NC * NS
BLK = 128
OCT = 8
DEPTH = 3


@functools.lru_cache(maxsize=None)
def _make_gather(hist: int, batch: int):
    nb = batch // BLK
    n_units = hist * nb
    per_w = n_units // NW
    assert per_w % OCT == 0 and per_w >= 2 * OCT
    n_oct = per_w // OCT
    mesh = plsc.VectorSubcoreMesh(core_axis_name="c", subcore_axis_name="s")

    @functools.partial(
        pl.kernel,
        mesh=mesh,
        out_type=jax.ShapeDtypeStruct((hist, EMB // 8, nb, 8, BLK), jnp.float32),
        scratch_types=[
            pltpu.VMEM((OCT * BLK,), jnp.int32),
            pltpu.VMEM((4, BLK), jnp.int32),
            pltpu.VMEM((4, BLK), jnp.int32),
            pltpu.VMEM((4, BLK, 128), jnp.float32),
            pltpu.VMEM((2, EMB, BLK), jnp.float32),
            pltpu.SemaphoreType.DMA,
            pltpu.SemaphoreType.DMA,
            pltpu.SemaphoreType.DMA,
            pltpu.SemaphoreType.DMA,
            pltpu.SemaphoreType.DMA,
            pltpu.SemaphoreType.DMA,
        ],
        compiler_params=pltpu.CompilerParams(
            use_tc_tiling_on_sc=True, needs_layout_passes=False
        ),
    )
    def gather_kernel(idx_hbm, tbl_hbm, out_hbm, ibuf, j_v, pb_v, gbuf, obuf,
                      sg0, sg1, sg2, sg3, so0, so1):
        wid = lax.axis_index("s") * NC + lax.axis_index("c")
        u0 = wid * per_w
        sem_g = (sg0, sg1, sg2, sg3)
        sem_o = (so0, so1)
        lanes = [lax.broadcasted_iota(jnp.int32, (16,), 0) + g * 16
                 for g in range(8)]

        def stage(o):
            u = u0 + o * OCT
            h = u // nb
            tb = u % nb
            pltpu.sync_copy(idx_hbm.at[h, pl.ds(tb * BLK, OCT * BLK)], ibuf)

        def gather_cp(b):
            return pltpu.make_async_copy(tbl_hbm.at[j_v.at[b]], gbuf.at[b],
                                         sem_g[b])

        def out_cps(v, b2):
            u = u0 + v
            h = u // nb
            tb = u % nb
            return [
                pltpu.make_async_copy(obuf.at[b2, pl.ds(te * 8, 8)],
                                      out_hbm.at[h, te, tb], sem_o[b2])
                for te in range(8)
            ]

        def prep(k, b):
            for g in range(8):
                iv = ibuf[pl.ds(k * BLK + g * 16, 16)]
                j_v[b, pl.ds(g * 16, 16)] = iv >> 1
                pb_v[b, pl.ds(g * 16, 16)] = (iv & 1) * 64
            gather_cp(b).start()

        def unit(v, b, b2, first):
            gather_cp(b).wait()
            if not first:
                for cp in out_cps(v, b2):
                    cp.wait()
            pbase = [pb_v[b, pl.ds(g * 16, 16)] for g in range(8)]

            @plsc.parallel_loop(0, EMB, unroll=4)
            def e_body(e):
                for g in range(8):
                    col = pbase[g] + jnp.full((16,), e, jnp.int32)
                    val = plsc.load_gather(gbuf.at[b], [lanes[g], col])
                    obuf[b2, e, pl.ds(g * 16, 16)] = val

            for cp in out_cps(v, b2):
                cp.start()

        stage(0)
        for k in range(DEPTH):
            prep(k, k % 4)
        for k in range(DEPTH, OCT):
            prep(k, k % 4)
            r = k - DEPTH
            unit(r, r % 4, r % 2, first=(r < 2))

        def oct_body(o, carry):
            v0 = o * OCT
            stage(o)
            for k in range(OCT):
                prep(k, k % 4)
                r = k - DEPTH
                unit(v0 + r, r % 4, r % 2, first=False)
            return carry

        lax.fori_loop(1, n_oct, oct_body, 0)

        for d in range(DEPTH, 0, -1):
            v = per_w - d
            unit(v, v % 4, v % 2, first=False)
        for cp in out_cps(per_w - 2, 0):
            cp.wait()
        for cp in out_cps(per_w - 1, 1):
            cp.wait()

    return gather_kernel


def kernel(input, table):
    batch, hist = input.shape
    vocab, emb = table.shape
    idxT = input.T.astype(jnp.int32)
    tbl2 = table.reshape(vocab // 2, 2 * emb)
    fn = _make_gather(hist, batch)
    out5 = fn(idxT, tbl2)
    return out5.transpose(2, 4, 0, 1, 3).reshape(batch, hist, emb)


# speedup vs baseline: 1.0030x; 1.0030x over previous
import functools

import jax
import jax.numpy as jnp
from jax import lax
from jax.experimental import pallas as pl
from jax.experimental.pallas import tpu as pltpu
from jax.experimental.pallas import tpu_sc as plsc

EMB = 64
NC = 2
NS = 16
NW = NC * NS
BLK = 128
OCT = 8
DEPTH = 3


@functools.lru_cache(maxsize=None)
def _make_gather(hist: int, batch: int):
    nb = batch // BLK
    n_units = hist * nb
    per_w = n_units // NW
    assert per_w % OCT == 0 and per_w >= 2 * OCT
    n_oct = per_w // OCT
    mesh = plsc.VectorSubcoreMesh(core_axis_name="c", subcore_axis_name="s")

    @functools.partial(
        pl.kernel,
        mesh=mesh,
        out_type=jax.ShapeDtypeStruct((hist, EMB // 8, nb, 8, BLK), jnp.float32),
        scratch_types=[
            pltpu.VMEM((OCT * BLK,), jnp.int32),
            pltpu.VMEM((4, BLK), jnp.int32),
            pltpu.VMEM((4, BLK), jnp.int32),
            pltpu.VMEM((4, BLK, 128), jnp.float32),
            pltpu.VMEM((2, EMB, BLK), jnp.float32),
            pltpu.SemaphoreType.DMA,
            pltpu.SemaphoreType.DMA,
            pltpu.SemaphoreType.DMA,
            pltpu.SemaphoreType.DMA,
            pltpu.SemaphoreType.DMA,
            pltpu.SemaphoreType.DMA,
        ],
        compiler_params=pltpu.CompilerParams(
            use_tc_tiling_on_sc=True, needs_layout_passes=False
        ),
    )
    def gather_kernel(idx_hbm, tbl_hbm, out_hbm, ibuf, j_v, pb_v, gbuf, obuf,
                      sg0, sg1, sg2, sg3, so0, so1):
        wid = lax.axis_index("s") * NC + lax.axis_index("c")
        u0 = wid * per_w
        sem_g = (sg0, sg1, sg2, sg3)
        sem_o = (so0, so1)
        lanes = [lax.broadcasted_iota(jnp.int32, (16,), 0) + g * 16
                 for g in range(8)]

        def stage(o):
            u = u0 + o * OCT
            h = u // nb
            tb = u % nb
            pltpu.sync_copy(idx_hbm.at[h, pl.ds(tb * BLK, OCT * BLK)], ibuf)

        def gather_cp(b):
            return pltpu.make_async_copy(tbl_hbm.at[j_v.at[b]], gbuf.at[b],
                                         sem_g[b])

        def out_cps(v, b2):
            u = u0 + v
            h = u // nb
            tb = u % nb
            return [
                pltpu.make_async_copy(obuf.at[b2, pl.ds(te * 8, 8)],
                                      out_hbm.at[h, te, tb], sem_o[b2])
                for te in range(8)
            ]

        def prep(k, b):
            for g in range(8):
                iv = ibuf[pl.ds(k * BLK + g * 16, 16)]
                j_v[b, pl.ds(g * 16, 16)] = iv >> 1
                pb_v[b, pl.ds(g * 16, 16)] = (iv & 1) * 64
            gather_cp(b).start()

        def unit(v, b, b2, first):
            gather_cp(b).wait()
            if not first:
                for cp in out_cps(v, b2):
                    cp.wait()
            pbase = [pb_v[b, pl.ds(g * 16, 16)] for g in range(8)]

            @plsc.parallel_loop(0, EMB, unroll=8)
            def e_body(e):
                for g in range(8):
                    col = pbase[g] + jnp.full((16,), e, jnp.int32)
                    val = plsc.load_gather(gbuf.at[b], [lanes[g], col])
                    obuf[b2, e, pl.ds(g * 16, 16)] = val

            for cp in out_cps(v, b2):
                cp.start()

        stage(0)
        for k in range(DEPTH):
            prep(k, k % 4)
        for k in range(DEPTH, OCT):
            prep(k, k % 4)
            r = k - DEPTH
            unit(r, r % 4, r % 2, first=(r < 2))

        def oct_body(o, carry):
            v0 = o * OCT
            stage(o)
            for k in range(OCT):
                prep(k, k % 4)
                r = k - DEPTH
                unit(v0 + r, r % 4, r % 2, first=False)
            return carry

        lax.fori_loop(1, n_oct, oct_body, 0)

        for d in range(DEPTH, 0, -1):
            v = per_w - d
            unit(v, v % 4, v % 2, first=False)
        for cp in out_cps(per_w - 2, 0):
            cp.wait()
        for cp in out_cps(per_w - 1, 1):
            cp.wait()

    return gather_kernel


def kernel(input, table):
    batch, hist = input.shape
    vocab, emb = table.shape
    idxT = input.T.astype(jnp.int32)
    tbl2 = table.reshape(vocab // 2, 2 * emb)
    fn = _make_gather(hist, batch)
    out5 = fn(idxT, tbl2)
    return out5.transpose(2, 4, 0, 1, 3).reshape(batch, hist, emb)
